# Initial kernel scaffold; baseline (speedup 1.0000x reference)
import jax
import jax.numpy as jnp
from jax.experimental import pallas as pl


def kernel(x, edge_index, mask, center_node_index, graph_id_index, W0, b0, gamma, beta, ge_W1, ge_b1, ge_W2, ge_b2, initial_embed, g_W1, g_b1, g_W2, g_b2, l1_W, l1_b, l2_W, l2_b):
    raise NotImplementedError("write your pallas kernel here")


# speedup vs baseline: 1.1037x; 1.1037x over previous
import jax
import jax.numpy as jnp
from jax import lax
from jax.experimental import pallas as pl
from jax.experimental.pallas import tpu as pltpu
from jax.experimental.pallas import tpu_sc as plsc

N = 10000
D = 128
G = 100
C = 100
E = 320000
NACC = 10240
EPAD = 327680
EPW = EPAD // 32
EB = 128
NB = EPW // EB
RPS = NACC // 16
IW = 16
BLK = 400
GRID = N // BLK


def _sc_mesh():
    return plsc.VectorSubcoreMesh(core_axis_name="c", subcore_axis_name="s",
                                  num_cores=2, num_subcores=16)


def _edge_pass_body(table_h, src_h, dst_h, zeros_h, out_h,
                    src_v, dst_v, rows_v, acc_s, gsem):
    c = lax.axis_index("c")
    s = lax.axis_index("s")

    @pl.when(s == 0)
    def _():
        pltpu.sync_copy(zeros_h, acc_s)

        def body(k, carry):
            base = (2 * (k // NB) + c) * EPW + (k % NB) * EB
            pltpu.sync_copy(src_h.at[pl.ds(base, EB)], src_v)
            pltpu.sync_copy(dst_h.at[pl.ds(base, EB)], dst_v)
            pltpu.async_copy(table_h.at[src_v], rows_v, gsem).wait()
            pltpu.sync_copy(rows_v, acc_s.at[dst_v], add=True)
            return carry

        lax.fori_loop(0, 16 * NB, body, 0)
        pltpu.sync_copy(acc_s, out_h.at[c])


def _edge_pass(table, src2d, dst2d, zeros):
    return pl.kernel(
        _edge_pass_body,
        out_type=jax.ShapeDtypeStruct((2, NACC, D), jnp.float32),
        mesh=_sc_mesh(),
        scratch_types=[
            pltpu.VMEM((EB,), jnp.int32),
            pltpu.VMEM((EB,), jnp.int32),
            pltpu.VMEM((EB, D), jnp.float32),
            pltpu.VMEM_SHARED((NACC, D), jnp.float32),
            pltpu.SemaphoreType.DMA,
        ],
    )(table, src2d, dst2d, zeros)


def _ones_pass_body(dst_h, zi_h, ones_h, dst_v, ones_v, ind_s, ind_h):
    c = lax.axis_index("c")
    s = lax.axis_index("s")

    @pl.when(s == 0)
    def _():
        pltpu.sync_copy(zi_h, ind_s)
        pltpu.sync_copy(ones_h, ones_v)

        def body(k, carry):
            base = (2 * (k // NB) + c) * EPW + (k % NB) * EB
            pltpu.sync_copy(dst_h.at[pl.ds(base, EB)], dst_v)
            pltpu.sync_copy(ones_v, ind_s.at[dst_v], add=True)
            return carry

        lax.fori_loop(0, 16 * NB, body, 0)
        pltpu.sync_copy(ind_s, ind_h.at[c])


def _ones_pass(dst2d, zi, ones):
    def body(dst_h, zi_h, ones_h, ind_h, dst_v, ones_v, ind_s):
        _ones_pass_body(dst_h, zi_h, ones_h, dst_v, ones_v, ind_s, ind_h)

    return pl.kernel(
        body,
        out_type=jax.ShapeDtypeStruct((2, NACC, IW), jnp.float32),
        mesh=_sc_mesh(),
        scratch_types=[
            pltpu.VMEM((EB,), jnp.int32),
            pltpu.VMEM((EB, IW), jnp.float32),
            pltpu.VMEM_SHARED((NACC, IW), jnp.float32),
        ],
    )(dst2d, zi, ones)


def _center_gather_body(enc_h, idx_h, out_h, idx_v, rows_v, sem):
    c = lax.axis_index("c")
    s = lax.axis_index("s")

    @pl.when(jnp.logical_and(c == 0, s == 0))
    def _():
        pltpu.sync_copy(idx_h, idx_v)
        pltpu.async_copy(enc_h.at[idx_v], rows_v, sem).wait()
        pltpu.sync_copy(rows_v, out_h)


def _center_gather(enc, cidx_pad):
    return pl.kernel(
        _center_gather_body,
        out_type=jax.ShapeDtypeStruct((128, 2 * D), jnp.float32),
        mesh=_sc_mesh(),
        scratch_types=[
            pltpu.VMEM((128,), jnp.int32),
            pltpu.VMEM((128, 2 * D), jnp.float32),
            pltpu.SemaphoreType.DMA,
        ],
    )(enc, cidx_pad)


def _ka_body(x_ref, w0_ref, b0_ref, y_ref, st_ref):
    i = pl.program_id(0)
    y = jnp.dot(x_ref[...], w0_ref[...],
                preferred_element_type=jnp.float32) + b0_ref[...]
    y_ref[...] = y
    st = jnp.concatenate([jnp.sum(y, 0, keepdims=True),
                          jnp.sum(y * y, 0, keepdims=True)], axis=0)

    @pl.when(i == 0)
    def _():
        st_ref[...] = jnp.zeros_like(st_ref)

    st_ref[...] += st


def _ka(x, w0, b0):
    return pl.pallas_call(
        _ka_body,
        grid=(GRID,),
        in_specs=[
            pl.BlockSpec((BLK, D), lambda i: (i, 0)),
            pl.BlockSpec((D, D), lambda i: (0, 0)),
            pl.BlockSpec((1, D), lambda i: (0, 0)),
        ],
        out_specs=[
            pl.BlockSpec((BLK, D), lambda i: (i, 0)),
            pl.BlockSpec((2, D), lambda i: (0, 0)),
        ],
        out_shape=[
            jax.ShapeDtypeStruct((N, D), jnp.float32),
            jax.ShapeDtypeStruct((2, D), jnp.float32),
        ],
    )(x, w0, b0)


def _kb_body(y_ref, p0_ref, p1_ref, i0_ref, i1_ref, st_ref, gam_ref, bet_ref,
             w1_ref, b1_ref, w2_ref, b2_ref, x0_ref, ind_ref):
    mu = st_ref[0:1, :] / N
    var = st_ref[1:2, :] / N - mu * mu
    sc = gam_ref[...] * lax.rsqrt(var + 1e-5)
    y = y_ref[...]
    indeg = i0_ref[...] + i1_ref[...]
    ind_ref[...] = indeg
    aggy = p0_ref[...] + p1_ref[...]
    xbn = (y - mu) * sc + bet_ref[...]
    agg1 = (aggy - indeg * mu) * sc + indeg * bet_ref[...]
    h = xbn + agg1
    t = jax.nn.relu(jnp.dot(h, w1_ref[...],
                            preferred_element_type=jnp.float32) + b1_ref[...])
    x0_ref[...] = jnp.dot(t, w2_ref[...],
                          preferred_element_type=jnp.float32) + b2_ref[...]


def _kb(y, p0, p1, i0, i1, st, gam, bet, w1, b1, w2, b2):
    return pl.pallas_call(
        _kb_body,
        grid=(GRID,),
        in_specs=[
            pl.BlockSpec((BLK, D), lambda i: (i, 0)),
            pl.BlockSpec((BLK, D), lambda i: (i, 0)),
            pl.BlockSpec((BLK, D), lambda i: (i, 0)),
            pl.BlockSpec((BLK, 1), lambda i: (i, 0)),
            pl.BlockSpec((BLK, 1), lambda i: (i, 0)),
            pl.BlockSpec((2, D), lambda i: (0, 0)),
            pl.BlockSpec((1, D), lambda i: (0, 0)),
            pl.BlockSpec((1, D), lambda i: (0, 0)),
            pl.BlockSpec((D, D), lambda i: (0, 0)),
            pl.BlockSpec((1, D), lambda i: (0, 0)),
            pl.BlockSpec((D, D), lambda i: (0, 0)),
            pl.BlockSpec((1, D), lambda i: (0, 0)),
        ],
        out_specs=[
            pl.BlockSpec((BLK, D), lambda i: (i, 0)),
            pl.BlockSpec((BLK, 1), lambda i: (i, 0)),
        ],
        out_shape=[
            jax.ShapeDtypeStruct((N, D), jnp.float32),
            jax.ShapeDtypeStruct((N, 1), jnp.float32),
        ],
    )(y, p0, p1, i0, i1, st, gam, bet, w1, b1, w2, b2)


def _kc_body(x0_ref, q0_ref, q1_ref, ind_ref, ie_ref,
             w1a_ref, w1b_ref, b1_ref, w2_ref, b2_ref, enc_ref):
    u = jnp.dot(ie_ref[...], w1a_ref[...],
                preferred_element_type=jnp.float32)
    indeg = ind_ref[...]
    h2b = x0_ref[...] + q0_ref[...] + q1_ref[...]
    z = jax.nn.relu((1.0 + indeg) * u +
                    jnp.dot(h2b, w1b_ref[...],
                            preferred_element_type=jnp.float32) + b1_ref[...])
    enc_ref[...] = jnp.dot(z, w2_ref[...],
                           preferred_element_type=jnp.float32) + b2_ref[...]


def _kc(x0, q0, q1, ind, ie, w1a, w1b, b1, w2, b2):
    return pl.pallas_call(
        _kc_body,
        grid=(GRID,),
        in_specs=[
            pl.BlockSpec((BLK, D), lambda i: (i, 0)),
            pl.BlockSpec((BLK, D), lambda i: (i, 0)),
            pl.BlockSpec((BLK, D), lambda i: (i, 0)),
            pl.BlockSpec((BLK, 1), lambda i: (i, 0)),
            pl.BlockSpec((1, D), lambda i: (0, 0)),
            pl.BlockSpec((D, 2 * D), lambda i: (0, 0)),
            pl.BlockSpec((D, 2 * D), lambda i: (0, 0)),
            pl.BlockSpec((1, 2 * D), lambda i: (0, 0)),
            pl.BlockSpec((2 * D, 2 * D), lambda i: (0, 0)),
            pl.BlockSpec((1, 2 * D), lambda i: (0, 0)),
        ],
        out_specs=pl.BlockSpec((BLK, 2 * D), lambda i: (i, 0)),
        out_shape=jax.ShapeDtypeStruct((N, 2 * D), jnp.float32),
    )(x0, q0, q1, ind, ie, w1a, w1b, b1, w2, b2)


def _kd_body(enc_ref, x_ref, cen_ref, oh_ref, a_ref, b_ref, l1b_ref,
             wa_ref, wb_ref, l2b_ref, m_ref, lg_ref):
    cen = cen_ref[0]
    cg = jnp.dot(cen, a_ref[...], preferred_element_type=jnp.float32)
    cgpart = jnp.dot(oh_ref[...], cg, preferred_element_type=jnp.float32)
    h = jax.nn.relu(cgpart +
                    jnp.dot(enc_ref[...], b_ref[...],
                            preferred_element_type=jnp.float32) + l1b_ref[...])
    lg = (jnp.dot(x_ref[...], wa_ref[...],
                  preferred_element_type=jnp.float32) +
          jnp.dot(h, wb_ref[...], preferred_element_type=jnp.float32) +
          l2b_ref[...] - (1.0 - m_ref[...]) * 1e6)
    lg_ref[...] = lg


def _kd(enc, x, cen3, oh, a, b, l1b, wa, wb, l2b, m):
    return pl.pallas_call(
        _kd_body,
        grid=(GRID,),
        in_specs=[
            pl.BlockSpec((BLK, 2 * D), lambda i: (i, 0)),
            pl.BlockSpec((BLK, D), lambda i: (i, 0)),
            pl.BlockSpec((1, 8, 2 * D), lambda i: (i, 0, 0)),
            pl.BlockSpec((BLK, 8), lambda i: (0, 0)),
            pl.BlockSpec((2 * D, D), lambda i: (0, 0)),
            pl.BlockSpec((2 * D, D), lambda i: (0, 0)),
            pl.BlockSpec((1, D), lambda i: (0, 0)),
            pl.BlockSpec((D, 1), lambda i: (0, 0)),
            pl.BlockSpec((D, 1), lambda i: (0, 0)),
            pl.BlockSpec((1, 1), lambda i: (0, 0)),
            pl.BlockSpec((BLK, 1), lambda i: (i, 0)),
        ],
        out_specs=pl.BlockSpec((BLK, 1), lambda i: (i, 0)),
        out_shape=jax.ShapeDtypeStruct((N, 1), jnp.float32),
    )(enc, x, cen3, oh, a, b, l1b, wa, wb, l2b, m)


def _ke_body(lg_ref, gid_ref, samp_ref, la_ref):
    lg = lg_ref[...]
    mx = jnp.max(lg, axis=1, keepdims=True)
    ex = jnp.exp(lg - mx)
    sm = jnp.sum(ex, axis=1, keepdims=True)
    probs = ex / sm
    samp = jnp.argmax(probs, axis=1).astype(jnp.int32)
    la = jnp.log(jnp.max(probs, axis=1))
    samp_ref[...] = samp[:, None] + gid_ref[...]
    la_ref[...] = la[:, None]


def _ke(lg2, gid2):
    return pl.pallas_call(
        _ke_body,
        out_shape=[
            jax.ShapeDtypeStruct((G, 1), jnp.int32),
            jax.ShapeDtypeStruct((G, 1), jnp.float32),
        ],
    )(lg2, gid2)


def kernel(x, edge_index, mask, center_node_index, graph_id_index,
           W0, b0, gamma, beta,
           ge_W1, ge_b1, ge_W2, ge_b2,
           initial_embed,
           g_W1, g_b1, g_W2, g_b2,
           l1_W, l1_b, l2_W, l2_b):
    f32 = jnp.float32
    i32 = jnp.int32
    src = edge_index[0].astype(i32)
    dst = edge_index[1].astype(i32)
    srcp = jnp.concatenate([src, jnp.zeros((EPAD - E,), i32)])
    dstp = jnp.concatenate([dst, jnp.full((EPAD - E,), NACC - 1, i32)])
    zeros128 = jnp.zeros((NACC, D), f32)
    maskf = mask.astype(f32).reshape(N, 1)
    cidxp = jnp.concatenate([center_node_index.astype(i32),
                             jnp.zeros((128 - G,), i32)])
    oh = (jnp.arange(BLK, dtype=i32)[:, None] // C ==
          jnp.arange(8, dtype=i32)[None, :]).astype(f32)
    gid2 = graph_id_index.astype(i32).reshape(G, 1)

    ones_blk = jnp.ones((EB, IW), f32)
    zeros_ind = jnp.zeros((NACC, IW), f32)

    y, stats = _ka(x, W0, b0.reshape(1, D))
    p0 = jax.ops.segment_sum(y[src], dst, num_segments=N)
    p1 = jnp.zeros_like(p0)
    i0 = jax.ops.segment_sum(jnp.ones((E, 1), f32), dst, num_segments=N)
    i1 = jnp.zeros_like(i0)
    x0, indeg = _kb(y, p0, p1, i0, i1, stats,
                    gamma.reshape(1, D), beta.reshape(1, D),
                    ge_W1, ge_b1.reshape(1, D), ge_W2, ge_b2.reshape(1, D))
    q0 = jax.ops.segment_sum(x0[src], dst, num_segments=N)
    q1 = jnp.zeros_like(q0)
    enc = _kc(x0, q0, q1, indeg, initial_embed.reshape(1, D),
              g_W1[:D, :], g_W1[D:, :], g_b1.reshape(1, 2 * D),
              g_W2, g_b2.reshape(1, 2 * D))
    cen = _center_gather(enc, cidxp)
    cen3 = jnp.concatenate([cen.reshape(32, 4, 2 * D),
                            jnp.zeros((32, 4, 2 * D), f32)], axis=1)
    lg = _kd(enc, x, cen3, oh, l1_W[:2 * D, :], l1_W[2 * D:, :],
             l1_b.reshape(1, D), l2_W[:D, :], l2_W[D:, :],
             l2_b.reshape(1, 1), maskf)
    lgp = jnp.pad(lg.reshape(G, C), ((0, 0), (0, 128 - C)),
                  constant_values=-1e9)
    samp2, la2 = _ke(lgp, gid2)
    return (samp2[:, 0], la2[:, 0], x0)
